# baseline (speedup 1.0000x reference)
import functools

import jax
import jax.numpy as jnp
from jax import lax
from jax.experimental import pallas as pl
from jax.experimental.pallas import tpu as pltpu
from jax.experimental.pallas import tpu_sc as plsc

N_ENTITIES = 1000000
RANK = 64
N_CLAUSES = 2
N_VARS = 2
N_CONSTRAINTS = 16384

NC = 2
NS = 16
L = 16
NW = NC * NS

TOTAL = N_CLAUSES * N_CONSTRAINTS
PER_W = TOTAL // NW
CH = 128
NCH = PER_W // CH
W_PER_CLAUSE = NW // N_CLAUSES


@functools.partial(
    pl.kernel,
    mesh=plsc.VectorSubcoreMesh(core_axis_name="c", subcore_axis_name="s"),
    compiler_params=pltpu.CompilerParams(
        needs_layout_passes=False, use_tc_tiling_on_sc=False),
    out_type=jax.ShapeDtypeStruct((NW, L), jnp.float32),
    scratch_types=[
        pltpu.VMEM((PER_W,), jnp.int32),
        pltpu.VMEM((PER_W,), jnp.int32),
        pltpu.VMEM((2, CH, RANK), jnp.float32),
        pltpu.VMEM((2, CH, RANK), jnp.float32),
        pltpu.VMEM((RANK, L), jnp.float32),
        pltpu.VMEM((L,), jnp.float32),
        pltpu.SemaphoreType.DMA,
        pltpu.SemaphoreType.DMA,
        pltpu.SemaphoreType.DMA,
        pltpu.SemaphoreType.DMA,
    ],
)
def _adv_sc(tab_hbm, idxs_hbm, idxo_hbm, db_hbm, out_hbm,
            idxs_v, idxo_v, sbuf, obuf, d_v, out_v,
            sem_s0, sem_s1, sem_o0, sem_o1):
    wid = lax.axis_index("s") * NC + lax.axis_index("c")
    base = wid * PER_W
    clause = wid // W_PER_CLAUSE

    pltpu.sync_copy(idxs_hbm.at[pl.ds(base, PER_W)], idxs_v)
    pltpu.sync_copy(idxo_hbm.at[pl.ds(base, PER_W)], idxo_v)
    pltpu.sync_copy(db_hbm.at[clause], d_v)

    sem_s = (sem_s0, sem_s1)
    sem_o = (sem_o0, sem_o1)

    def start(g, b):
        off = g * CH
        cps = (
            pltpu.async_copy(tab_hbm.at[idxs_v.at[pl.ds(off, CH)]],
                             sbuf.at[b], sem_s[b]),
            pltpu.async_copy(tab_hbm.at[idxo_v.at[pl.ds(off, CH)]],
                             obuf.at[b], sem_o[b]),
        )
        return cps

    def compute(b, acc):
        sref = sbuf.at[b]
        oref = obuf.at[b]

        def group_body(gg, acc):
            rows = lax.iota(jnp.int32, 16) + gg * L

            def r_body(r, score):
                cols = jnp.full((L,), r, jnp.int32)
                sv = plsc.load_gather(sref, [rows, cols])
                ov = plsc.load_gather(oref, [rows, cols])
                dv = d_v[r, :]
                return score + sv * ov * dv

            score = lax.fori_loop(0, RANK, r_body,
                                  jnp.zeros((L,), jnp.float32), unroll=8)
            return acc + jnp.maximum(score, 0.0)

        return lax.fori_loop(0, CH // L, group_body, acc)

    cps = [None, None]
    cps[0] = start(0, 0)
    acc = jnp.zeros((L,), jnp.float32)
    for g in range(NCH):
        b = g & 1
        if g + 1 < NCH:
            cps[(g + 1) & 1] = start(g + 1, (g + 1) & 1)
        cps[b][0].wait()
        cps[b][1].wait()
        acc = compute(b, acc)

    out_v[...] = acc
    pltpu.sync_copy(out_v, out_hbm.at[wid])


def kernel(emb_so, rel, adv_indices):
    idx = adv_indices.astype(jnp.int32)
    idx_s = idx[:, 0, :].reshape(-1)
    idx_o = idx[:, 1, :].reshape(-1)
    d = rel[:, 0, :] - rel[:, 1, :]
    db = jnp.broadcast_to(d[:, :, None], (N_CLAUSES, RANK, L))
    partials = _adv_sc(emb_so, idx_s, idx_o, db)
    return jnp.sum(partials)


# speedup vs baseline: 1.5682x; 1.5682x over previous
import functools

import jax
import jax.numpy as jnp
from jax import lax
from jax.experimental import pallas as pl
from jax.experimental.pallas import tpu as pltpu
from jax.experimental.pallas import tpu_sc as plsc

N_ENTITIES = 1000000
RANK = 64
N_CLAUSES = 2
N_VARS = 2
N_CONSTRAINTS = 16384

NC = 2
NS = 16
L = 16
NW = NC * NS

TOTAL = N_CLAUSES * N_CONSTRAINTS
PER_W = TOTAL // NW
CH = 128
NCH = PER_W // CH
W_PER_CLAUSE = NW // N_CLAUSES
BUF = CH * RANK


@functools.partial(
    pl.kernel,
    mesh=plsc.VectorSubcoreMesh(core_axis_name="c", subcore_axis_name="s"),
    compiler_params=pltpu.CompilerParams(
        needs_layout_passes=False, use_tc_tiling_on_sc=True),
    out_type=jax.ShapeDtypeStruct((NW * L,), jnp.float32),
    scratch_types=[
        pltpu.VMEM((2, CH), jnp.int32),
        pltpu.VMEM((2, CH), jnp.int32),
        pltpu.VMEM((2, CH, RANK), jnp.float32),
        pltpu.VMEM((2, CH, RANK), jnp.float32),
        pltpu.VMEM((RANK * L,), jnp.float32),
        pltpu.VMEM((L,), jnp.float32),
        pltpu.SemaphoreType.DMA,
        pltpu.SemaphoreType.DMA,
        pltpu.SemaphoreType.DMA,
        pltpu.SemaphoreType.DMA,
    ],
)
def _adv_sc(tab_hbm, idxs_hbm, idxo_hbm, db_hbm, out_hbm,
            idxs_sm, idxo_sm, sbuf, obuf, d_v, out_v,
            sem_s0, sem_s1, sem_o0, sem_o1):
    wid = lax.axis_index("s") * NC + lax.axis_index("c")
    base = wid * PER_W
    clause = wid // W_PER_CLAUSE

    pltpu.sync_copy(db_hbm.at[pl.ds(clause * RANK * L, RANK * L)], d_v)

    sem_s = (sem_s0, sem_s1)
    sem_o = (sem_o0, sem_o1)

    def start(g, b):
        off = base + g * CH
        pltpu.sync_copy(idxs_hbm.at[pl.ds(off, CH)], idxs_sm.at[b])
        pltpu.sync_copy(idxo_hbm.at[pl.ds(off, CH)], idxo_sm.at[b])

        def issue(gg, _):
            jb = gg * L
            vs = idxs_sm[b, pl.ds(jb, L)]
            vo = idxo_sm[b, pl.ds(jb, L)]
            for j in range(L):
                pltpu.async_copy(tab_hbm.at[vs[j]], sbuf.at[b, jb + j],
                                 sem_s[b])
                pltpu.async_copy(tab_hbm.at[vo[j]], obuf.at[b, jb + j],
                                 sem_o[b])
            return 0

        lax.fori_loop(0, CH // L, issue, 0)

    def drain(b):
        pltpu.make_async_copy(
            tab_hbm.at[pl.ds(0, CH)], sbuf.at[b], sem_s[b]).wait()
        pltpu.make_async_copy(
            tab_hbm.at[pl.ds(0, CH)], obuf.at[b], sem_o[b]).wait()

    def compute(b, acc):
        sref = sbuf.at[b]
        oref = obuf.at[b]

        def group_body(gg, acc):
            rows = lax.iota(jnp.int32, 16) + gg * L

            def r_body(r, score):
                cols = jnp.full((L,), r, jnp.int32)
                sv = plsc.load_gather(sref, [rows, cols])
                ov = plsc.load_gather(oref, [rows, cols])
                dv = d_v[pl.ds(r * L, L)]
                return score + sv * ov * dv

            score = lax.fori_loop(0, RANK, r_body,
                                  jnp.zeros((L,), jnp.float32), unroll=8)
            return acc + jnp.maximum(score, 0.0)

        return lax.fori_loop(0, CH // L, group_body, acc)

    start(0, 0)
    acc = jnp.zeros((L,), jnp.float32)
    for g in range(NCH):
        b = g & 1
        if g + 1 < NCH:
            start(g + 1, (g + 1) & 1)
        drain(b)
        acc = compute(b, acc)

    out_v[...] = acc
    pltpu.sync_copy(out_v, out_hbm.at[pl.ds(wid * L, L)])


def kernel(emb_so, rel, adv_indices):
    idx = adv_indices.astype(jnp.int32)
    idx_s = idx[:, 0, :].reshape(-1)
    idx_o = idx[:, 1, :].reshape(-1)
    d = rel[:, 0, :] - rel[:, 1, :]
    db = jnp.broadcast_to(d[:, :, None], (N_CLAUSES, RANK, L))
    db = db.reshape(N_CLAUSES * RANK * L)
    partials = _adv_sc(emb_so, idx_s, idx_o, db)
    return jnp.sum(partials)
